# baseline (speedup 1.0000x reference)
import functools

import jax
import jax.numpy as jnp
from jax import lax
from jax.experimental import pallas as pl
from jax.experimental.pallas import tpu as pltpu
from jax.experimental.pallas import tpu_sc as plsc

_DIM = 1024
_NTOK = 4 * 4096
_GRP = 2
_DG = _GRP * _DIM
_NUNIT = _NTOK // _GRP
_NCOMBO = 3 ** _GRP
_NC = 2
_NS = 16
_NW = _NC * _NS
_UPW = _NUNIT // _NW
_CHUNK = 16
_NCHUNK = _UPW // _CHUNK

_mesh = plsc.VectorSubcoreMesh(core_axis_name="c", subcore_axis_name="s")


@functools.partial(
    pl.kernel,
    mesh=_mesh,
    out_type=jax.ShapeDtypeStruct((_NUNIT, _DG), jnp.float32),
    scratch_types=[
        pltpu.VMEM((_NCHUNK, _CHUNK), jnp.int32),
        pltpu.VMEM((_CHUNK, _DG), jnp.float32),
        pltpu.VMEM((_CHUNK, _DG), jnp.float32),
        pltpu.SemaphoreType.DMA,
        pltpu.SemaphoreType.DMA,
        pltpu.SemaphoreType.DMA,
        pltpu.SemaphoreType.DMA,
    ],
)
def _emb_lookup(idx_hbm, w_hbm, out_hbm, idx_v, rows0, rows1, g0, g1, s0, s1):
    wid = lax.axis_index("s") * _NC + lax.axis_index("c")
    base = wid * _UPW
    pltpu.sync_copy(idx_hbm.at[wid], idx_v)
    rows = (rows0, rows1)
    gsem = (g0, g1)
    ssem = (s0, s1)
    gh = [None] * _NCHUNK
    sh = [None] * _NCHUNK
    gh[0] = pltpu.async_copy(w_hbm.at[idx_v.at[0]], rows0, g0)
    for c in range(_NCHUNK):
        b = c & 1
        gh[c].wait()
        if c + 1 < _NCHUNK:
            if c >= 1:
                sh[c - 1].wait()
            gh[c + 1] = pltpu.async_copy(
                w_hbm.at[idx_v.at[c + 1]], rows[1 - b], gsem[1 - b]
            )
        sh[c] = pltpu.async_copy(
            rows[b], out_hbm.at[pl.ds(base + c * _CHUNK, _CHUNK)], ssem[b]
        )
    sh[_NCHUNK - 2].wait()
    sh[_NCHUNK - 1].wait()


def kernel(indices, weight):
    flat = indices.reshape(-1).astype(jnp.int32)
    pair = flat.reshape(_NUNIT, _GRP)
    pid = pair[:, 0] * 3 + pair[:, 1]
    idx = pid.reshape(_NW, _NCHUNK, _CHUNK)
    offs = (_NCOMBO * jnp.arange(_NW, dtype=jnp.int32))[:, None, None]
    combo = jnp.arange(_NCOMBO, dtype=jnp.int32)
    pair_tab = jnp.concatenate(
        [jnp.take(weight, combo // 3, axis=0), jnp.take(weight, combo % 3, axis=0)],
        axis=1,
    )
    w_rep = jnp.tile(pair_tab, (_NW, 1))
    out = _emb_lookup(idx + offs, w_rep)
    return out.reshape(indices.shape[0], indices.shape[1], _DIM)


# speedup vs baseline: 1.4796x; 1.4796x over previous
import functools

import jax
import jax.numpy as jnp
from jax import lax
from jax.experimental import pallas as pl
from jax.experimental.pallas import tpu as pltpu
from jax.experimental.pallas import tpu_sc as plsc

_DIM = 1024
_NTOK = 4 * 4096
_NC = 2
_NS = 16
_NW = _NC * _NS
_TPW = _NTOK // _NW
_CHUNK = 16
_NCHUNK = _TPW // _CHUNK
_NBUF = 4
_RSTRIDE = 4

_mesh = plsc.VectorSubcoreMesh(core_axis_name="c", subcore_axis_name="s")

_scratch = [pltpu.VMEM((_NCHUNK, _CHUNK), jnp.int32)]
_scratch += [pltpu.VMEM((_CHUNK, _DIM), jnp.float32) for _ in range(_NBUF)]
_scratch += [pltpu.SemaphoreType.DMA for _ in range(2 * _NBUF)]


@functools.partial(
    pl.kernel,
    mesh=_mesh,
    out_type=jax.ShapeDtypeStruct((_NTOK, _DIM), jnp.float32),
    scratch_types=_scratch,
)
def _emb_lookup(idx_hbm, w_hbm, out_hbm, idx_v, *bufs_sems):
    rows = bufs_sems[:_NBUF]
    gsem = bufs_sems[_NBUF : 2 * _NBUF]
    ssem = bufs_sems[2 * _NBUF :]
    wid = lax.axis_index("s") * _NC + lax.axis_index("c")
    base = wid * _TPW
    pltpu.sync_copy(idx_hbm.at[wid], idx_v)
    gh = [None] * _NCHUNK
    sh = [None] * _NCHUNK
    for c in range(_NBUF - 1):
        gh[c] = pltpu.async_copy(w_hbm.at[idx_v.at[c]], rows[c % _NBUF], gsem[c % _NBUF])
    for c in range(_NCHUNK):
        b = c % _NBUF
        gh[c].wait()
        n = c + _NBUF - 1
        if n < _NCHUNK:
            if c >= 1:
                sh[c - 1].wait()
            gh[n] = pltpu.async_copy(w_hbm.at[idx_v.at[n]], rows[n % _NBUF], gsem[n % _NBUF])
        sh[c] = pltpu.async_copy(
            rows[b], out_hbm.at[pl.ds(base + c * _CHUNK, _CHUNK)], ssem[b]
        )
    for c in range(_NCHUNK - _NBUF + 1, _NCHUNK):
        sh[c].wait()


def kernel(indices, weight):
    idx = indices.reshape(_NW, _NCHUNK, _CHUNK).astype(jnp.int32)
    offs = (_RSTRIDE * jnp.arange(_NW, dtype=jnp.int32))[:, None, None]
    w_rep = jnp.tile(
        jnp.concatenate(
            [weight, jnp.zeros((_RSTRIDE - 3, _DIM), jnp.float32)], axis=0
        ),
        (_NW, 1),
    )
    out = _emb_lookup(idx + offs, w_rep)
    return out.reshape(indices.shape[0], indices.shape[1], _DIM)
